# baseline (speedup 1.0000x reference)
import functools

import jax
import jax.numpy as jnp
from jax import lax
from jax.experimental import pallas as pl
from jax.experimental.pallas import tpu as pltpu
from jax.experimental.pallas import tpu_sc as plsc

_NC = 2
_NS = 16
_NW = _NC * _NS
_CHUNK = 32


def _make_sc_copy(rows, d, dtype):
    rows_per_w = rows // _NW
    nch = rows_per_w // _CHUNK
    mesh = plsc.VectorSubcoreMesh(core_axis_name="c", subcore_axis_name="s")

    @functools.partial(
        pl.kernel,
        mesh=mesh,
        out_type=jax.ShapeDtypeStruct((rows, d), dtype),
        scratch_types=[
            pltpu.VMEM((_CHUNK, d), dtype),
            pltpu.VMEM((_CHUNK, d), dtype),
            pltpu.SemaphoreType.DMA,
            pltpu.SemaphoreType.DMA,
            pltpu.SemaphoreType.DMA,
            pltpu.SemaphoreType.DMA,
        ],
    )
    def sc_copy(in_hbm, out_hbm, buf0, buf1, rs0, rs1, ws0, ws1):
        wid = lax.axis_index("s") * _NC + lax.axis_index("c")
        base = wid * rows_per_w
        bufs = (buf0, buf1)
        rsems = (rs0, rs1)
        wsems = (ws0, ws1)

        def rd(i):
            return pltpu.make_async_copy(
                in_hbm.at[pl.ds(base + i * _CHUNK, _CHUNK)], bufs[i % 2], rsems[i % 2])

        def wr(i):
            return pltpu.make_async_copy(
                bufs[i % 2], out_hbm.at[pl.ds(base + i * _CHUNK, _CHUNK)], wsems[i % 2])

        rd(0).start()
        for i in range(nch):
            if i + 1 < nch:
                if i >= 1:
                    wr(i - 1).wait()
                rd(i + 1).start()
            rd(i).wait()
            wr(i).start()
        if nch >= 2:
            wr(nch - 2).wait()
        wr(nch - 1).wait()

    return sc_copy


def kernel(sequences, batch_first, padding_value, padding_side):
    B, L, D = sequences.shape
    rows = B * L
    flat = sequences.reshape(rows, D)
    out = _make_sc_copy(rows, D, sequences.dtype)(flat)
    return out.reshape(B, L, D)
